# baseline (speedup 1.0000x reference)
import functools

import jax
import jax.numpy as jnp
from jax import lax
from jax.experimental import pallas as pl
from jax.experimental.pallas import tpu as pltpu
from jax.experimental.pallas import tpu_sc as plsc

B, N, D, K, E = 4, 4096, 128, 27, 256
NT = 256
T = N // NT
SG = 16
KP = 32


S = 6
NC_ = N // 128


def _topk_kernel(xv_ref, xvt_ref, out_ref, dmat_ref, flags_ref):
    b = pl.program_id(0)
    xq = xv_ref[0]
    xpt = xvt_ref[0]
    d2p = (xpt[0:1, :] * xpt[0:1, :] + xpt[1:2, :] * xpt[1:2, :]) + xpt[2:3, :] * xpt[2:3, :]

    base = b * N
    inf = jnp.float32(jnp.inf)
    bigi = jnp.int32(N)
    liota = lax.broadcasted_iota(jnp.int32, (SG, 128), 1)
    kiota = lax.broadcasted_iota(jnp.int32, (SG, KP), 1)

    def subgroup(rg):
        xq_sg = xq[rg * SG:(rg + 1) * SG, :]
        d2q = (
            xq_sg[:, 0:1] * xq_sg[:, 0:1] + xq_sg[:, 1:2] * xq_sg[:, 1:2]
        ) + xq_sg[:, 2:3] * xq_sg[:, 2:3]
        dot = jnp.dot(xq_sg, xpt, preferred_element_type=jnp.float32)
        dmat = (d2q + d2p) - 2.0 * dot
        dmat_ref[pl.ds(rg * SG, SG), :] = dmat
        v = [jnp.full((SG, 128), inf, jnp.float32) for _ in range(S)]
        gi = [jnp.full((SG, 128), bigi, jnp.int32) for _ in range(S)]
        for c in range(NC_):
            t = dmat[:, c * 128:(c + 1) * 128]
            ti = liota + (c * 128)
            for j in range(S):
                lt = t < v[j]
                nv = jnp.where(lt, t, v[j])
                ni = jnp.where(lt, ti, gi[j])
                if j < S - 1:
                    t = jnp.where(lt, v[j], t)
                    ti = jnp.where(lt, gi[j], ti)
                v[j], gi[j] = nv, ni
        res = (
            lax.broadcasted_iota(jnp.int32, (SG, KP), 0) + (rg * SG + base)
        )
        flag = jnp.zeros((SG, 128), jnp.int32)
        for k in range(K):
            m = jnp.min(v[0], axis=1, keepdims=True)
            ik = jnp.min(jnp.where(v[0] == m, gi[0], bigi), axis=1, keepdims=True)
            res = jnp.where(kiota == k, ik + base, res)
            ext = (v[0] == m) & (gi[0] == ik)
            for j in range(S - 1):
                v[j] = jnp.where(ext, v[j + 1], v[j])
                gi[j] = jnp.where(ext, gi[j + 1], gi[j])
            v[S - 1] = jnp.where(ext, inf, v[S - 1])
            gi[S - 1] = jnp.where(ext, bigi, gi[S - 1])
            flag = flag | jnp.where(ext & (v[0] == inf), 1, 0)
        out_ref[0, 0, pl.ds(rg * SG, SG), :] = res
        flags_ref[rg] = jnp.max(flag)

    for rg_ in range(NT // SG):
        subgroup(rg_)

    giota = lax.broadcasted_iota(jnp.int32, (SG, N), 1)

    def fb_subgroup(rg, _):
        @pl.when(flags_ref[rg] > 0)
        def _():
            def fb_k(k, res):
                d = dmat_ref[pl.ds(rg * SG, SG), :]
                m = jnp.min(d, axis=1, keepdims=True)
                ik = jnp.min(jnp.where(d == m, giota, bigi), axis=1, keepdims=True)
                dmat_ref[pl.ds(rg * SG, SG), :] = jnp.where(giota == ik, inf, d)
                return jnp.where(kiota == k, ik + base, res)

            res = lax.fori_loop(0, K, fb_k, jnp.zeros((SG, KP), jnp.int32))
            out_ref[0, 0, pl.ds(rg * SG, SG), :] = res

        return 0

    lax.fori_loop(0, NT // SG, fb_subgroup, 0)


def _run_topk(x_v):
    xvt = x_v.transpose(0, 2, 1)
    return pl.pallas_call(
        _topk_kernel,
        grid=(B, T),
        in_specs=[
            pl.BlockSpec((1, NT, 3), lambda b, t: (b, t, 0)),
            pl.BlockSpec((1, 3, N), lambda b, t: (b, 0, 0)),
        ],
        out_specs=pl.BlockSpec((1, 1, NT, KP), lambda b, t: (b, t, 0, 0)),
        out_shape=jax.ShapeDtypeStruct((B, T, NT, KP), jnp.int32),
        scratch_shapes=[
            pltpu.VMEM((NT, N), jnp.float32),
            pltpu.SMEM((NT // SG,), jnp.int32),
        ],
    )(x_v, xvt)


NW = 32
ROWS = B * N * KP
PW = ROWS // NW
CH = 256
NCH = PW // CH


def _gather_body(table_hbm, idx_hbm, out_hbm, idx_v, rows0, rows1, sem0, sem1):
    wid = lax.axis_index("s") * 2 + lax.axis_index("c")
    base = wid * PW
    pltpu.sync_copy(idx_hbm.at[pl.ds(base, PW)], idx_v)
    bufs = (rows0, rows1)
    sems = (sem0, sem1)
    pltpu.async_copy(table_hbm.at[idx_v.at[pl.ds(0, CH)]], rows0, sem0)
    for ci in range(NCH):
        if ci + 1 < NCH:
            pltpu.async_copy(
                table_hbm.at[idx_v.at[pl.ds((ci + 1) * CH, CH)]],
                bufs[(ci + 1) % 2],
                sems[(ci + 1) % 2],
            )
        pltpu.make_async_copy(
            table_hbm.at[idx_v.at[pl.ds(ci * CH, CH)]],
            bufs[ci % 2],
            sems[ci % 2],
        ).wait()
        pltpu.sync_copy(bufs[ci % 2], out_hbm.at[pl.ds(base + ci * CH, CH)])


@functools.cache
def _make_gather():
    return pl.kernel(
        _gather_body,
        mesh=plsc.VectorSubcoreMesh(core_axis_name="c", subcore_axis_name="s"),
        out_type=jax.ShapeDtypeStruct((ROWS, D), jnp.float32),
        scratch_types=[
            pltpu.VMEM((PW,), jnp.int32),
            pltpu.VMEM((CH, D), jnp.float32),
            pltpu.VMEM((CH, D), jnp.float32),
            pltpu.SemaphoreType.DMA,
            pltpu.SemaphoreType.DMA,
        ],
    )


def _mm_kernel(flat_ref, wt_ref, b_ref, out_ref):
    out_ref[...] = (
        jnp.dot(flat_ref[...], wt_ref[...], preferred_element_type=jnp.float32)
        + b_ref[...]
    )


def _run_mm(flat, wt, bias):
    return pl.pallas_call(
        _mm_kernel,
        grid=(B * T,),
        in_specs=[
            pl.BlockSpec((NT, KP * D), lambda i: (i, 0)),
            pl.BlockSpec((KP * D, E), lambda i: (0, 0)),
            pl.BlockSpec((1, E), lambda i: (0, 0)),
        ],
        out_specs=pl.BlockSpec((NT, E), lambda i: (i, 0)),
        out_shape=jax.ShapeDtypeStruct((B * T * NT, E), jnp.float32),
    )(flat, wt, bias)


def kernel(x, x_v, W, b):
    idxg = _run_topk(x_v)
    idx = idxg.reshape(ROWS)
    table = x.reshape(B * N, D)
    g = _make_gather()(table, idx)
    flat = g.reshape(B * T * NT, KP * D)
    wt = jnp.concatenate(
        [W.T, jnp.zeros(((KP - K) * D, E), jnp.float32)], axis=0
    )
    out = _run_mm(flat, wt, b.reshape(1, E))
    return out.reshape(B, N, E)


# speedup vs baseline: 1.0777x; 1.0777x over previous
import functools

import jax
import jax.numpy as jnp
from jax import lax
from jax.experimental import pallas as pl
from jax.experimental.pallas import tpu as pltpu
from jax.experimental.pallas import tpu_sc as plsc

B, N, D, K, E = 4, 4096, 128, 27, 256
NT = 256
T = N // NT
SG = 16
KP = 32


S = 6
NC_ = N // 128


def _topk_kernel(xv_ref, xvt_ref, out_ref, dmat_ref, flags_ref):
    b = pl.program_id(0)
    xq = xv_ref[0]
    xpt = xvt_ref[0]
    d2p = (xpt[0:1, :] * xpt[0:1, :] + xpt[1:2, :] * xpt[1:2, :]) + xpt[2:3, :] * xpt[2:3, :]

    base = b * N
    inf = jnp.float32(jnp.inf)
    bigi = jnp.int32(N)
    liota = lax.broadcasted_iota(jnp.int32, (SG, 128), 1)
    kiota = lax.broadcasted_iota(jnp.int32, (SG, KP), 1)

    def subgroup(rg):
        xq_sg = xq[rg * SG:(rg + 1) * SG, :]
        d2q = (
            xq_sg[:, 0:1] * xq_sg[:, 0:1] + xq_sg[:, 1:2] * xq_sg[:, 1:2]
        ) + xq_sg[:, 2:3] * xq_sg[:, 2:3]
        dot = jnp.dot(xq_sg, xpt, preferred_element_type=jnp.float32)
        dmat = (d2q + d2p) - 2.0 * dot
        dmat_ref[pl.ds(rg * SG, SG), :] = dmat
        v = [jnp.full((SG, 128), inf, jnp.float32) for _ in range(S)]
        gi = [jnp.full((SG, 128), bigi, jnp.int32) for _ in range(S)]
        for c in range(NC_):
            t = dmat[:, c * 128:(c + 1) * 128]
            ti = liota + (c * 128)
            for j in range(S):
                lt = t < v[j]
                nv = jnp.where(lt, t, v[j])
                ni = jnp.where(lt, ti, gi[j])
                if j < S - 1:
                    t = jnp.where(lt, v[j], t)
                    ti = jnp.where(lt, gi[j], ti)
                v[j], gi[j] = nv, ni
        res = (
            lax.broadcasted_iota(jnp.int32, (SG, KP), 0) + (rg * SG + base)
        )
        flag = jnp.zeros((SG, 128), jnp.int32)
        for k in range(K):
            m = jnp.min(v[0], axis=1, keepdims=True)
            ik = jnp.min(jnp.where(v[0] == m, gi[0], bigi), axis=1, keepdims=True)
            res = jnp.where(kiota == k, ik + base, res)
            ext = (v[0] == m) & (gi[0] == ik)
            for j in range(S - 1):
                v[j] = jnp.where(ext, v[j + 1], v[j])
                gi[j] = jnp.where(ext, gi[j + 1], gi[j])
            v[S - 1] = jnp.where(ext, inf, v[S - 1])
            gi[S - 1] = jnp.where(ext, bigi, gi[S - 1])
            flag = flag | jnp.where(ext & (v[0] == inf), 1, 0)
        out_ref[0, 0, pl.ds(rg * SG, SG), :] = res
        flags_ref[rg] = jnp.max(flag)

    for rg_ in range(NT // SG):
        subgroup(rg_)

    giota = lax.broadcasted_iota(jnp.int32, (SG, N), 1)

    def fb_subgroup(rg, _):
        @pl.when(flags_ref[rg] > 0)
        def _():
            def fb_k(k, res):
                d = dmat_ref[pl.ds(rg * SG, SG), :]
                m = jnp.min(d, axis=1, keepdims=True)
                ik = jnp.min(jnp.where(d == m, giota, bigi), axis=1, keepdims=True)
                dmat_ref[pl.ds(rg * SG, SG), :] = jnp.where(giota == ik, inf, d)
                return jnp.where(kiota == k, ik + base, res)

            res = lax.fori_loop(0, K, fb_k, jnp.zeros((SG, KP), jnp.int32))
            out_ref[0, 0, pl.ds(rg * SG, SG), :] = res

        return 0

    lax.fori_loop(0, NT // SG, fb_subgroup, 0)


def _run_topk(x_v):
    xvt = x_v.transpose(0, 2, 1)
    return pl.pallas_call(
        _topk_kernel,
        grid=(B, T),
        in_specs=[
            pl.BlockSpec((1, NT, 3), lambda b, t: (b, t, 0)),
            pl.BlockSpec((1, 3, N), lambda b, t: (b, 0, 0)),
        ],
        out_specs=pl.BlockSpec((1, 1, NT, KP), lambda b, t: (b, t, 0, 0)),
        out_shape=jax.ShapeDtypeStruct((B, T, NT, KP), jnp.int32),
        scratch_shapes=[
            pltpu.VMEM((NT, N), jnp.float32),
            pltpu.SMEM((NT // SG,), jnp.int32),
        ],
    )(x_v, xvt)


NW = 32
ROWS = B * N * K
PW = ROWS // NW
CH = 256
NCH = PW // CH


def _gather_body(table_hbm, idx_hbm, out_hbm, idx_v, rows0, rows1, sem0, sem1):
    wid = lax.axis_index("s") * 2 + lax.axis_index("c")
    base = wid * PW
    pltpu.sync_copy(idx_hbm.at[pl.ds(base, PW)], idx_v)
    bufs = (rows0, rows1)
    sems = (sem0, sem1)
    pltpu.async_copy(table_hbm.at[idx_v.at[pl.ds(0, CH)]], rows0, sem0)
    for ci in range(NCH):
        if ci + 1 < NCH:
            pltpu.async_copy(
                table_hbm.at[idx_v.at[pl.ds((ci + 1) * CH, CH)]],
                bufs[(ci + 1) % 2],
                sems[(ci + 1) % 2],
            )
        pltpu.make_async_copy(
            table_hbm.at[idx_v.at[pl.ds(ci * CH, CH)]],
            bufs[ci % 2],
            sems[ci % 2],
        ).wait()
        pltpu.sync_copy(bufs[ci % 2], out_hbm.at[pl.ds(base + ci * CH, CH)])


@functools.cache
def _make_gather():
    return pl.kernel(
        _gather_body,
        mesh=plsc.VectorSubcoreMesh(core_axis_name="c", subcore_axis_name="s"),
        out_type=jax.ShapeDtypeStruct((ROWS, D), jnp.float32),
        scratch_types=[
            pltpu.VMEM((PW,), jnp.int32),
            pltpu.VMEM((CH, D), jnp.float32),
            pltpu.VMEM((CH, D), jnp.float32),
            pltpu.SemaphoreType.DMA,
            pltpu.SemaphoreType.DMA,
        ],
    )


def _mm_kernel(flat_ref, wt_ref, b_ref, out_ref):
    out_ref[...] = (
        jnp.dot(flat_ref[...], wt_ref[...], preferred_element_type=jnp.float32)
        + b_ref[...]
    )


def _run_mm(flat, wt, bias):
    return pl.pallas_call(
        _mm_kernel,
        grid=(B * T,),
        in_specs=[
            pl.BlockSpec((NT, K * D), lambda i: (i, 0)),
            pl.BlockSpec((K * D, E), lambda i: (0, 0)),
            pl.BlockSpec((1, E), lambda i: (0, 0)),
        ],
        out_specs=pl.BlockSpec((NT, E), lambda i: (i, 0)),
        out_shape=jax.ShapeDtypeStruct((B * T * NT, E), jnp.float32),
    )(flat, wt, bias)


def kernel(x, x_v, W, b):
    idxg = _run_topk(x_v)
    idx = idxg[..., :K].reshape(ROWS)
    table = x.reshape(B * N, D)
    g = _make_gather()(table, idx)
    flat = g.reshape(B * T * NT, K * D)
    out = _run_mm(flat, W.T, b.reshape(1, E))
    return out.reshape(B, N, E)


# speedup vs baseline: 1.1273x; 1.0460x over previous
import functools

import jax
import jax.numpy as jnp
from jax import lax
from jax.experimental import pallas as pl
from jax.experimental.pallas import tpu as pltpu
from jax.experimental.pallas import tpu_sc as plsc

B, N, D, K, E = 4, 4096, 128, 27, 256
NT = 256
T = N // NT
SG = 16
KP = 32


S = 6
NC_ = N // 128


def _topk_kernel(xv_ref, xvt_ref, out_ref, dmat_ref, flags_ref):
    xq = xv_ref[...]
    xpt = xvt_ref[...]
    d2p = (xpt[0:1, :] * xpt[0:1, :] + xpt[1:2, :] * xpt[1:2, :]) + xpt[2:3, :] * xpt[2:3, :]

    base = 0
    inf = jnp.float32(jnp.inf)
    bigi = jnp.int32(N)
    liota = lax.broadcasted_iota(jnp.int32, (SG, 128), 1)
    kiota = lax.broadcasted_iota(jnp.int32, (SG, KP), 1)

    def subgroup(rg):
        xq_sg = xq[rg * SG:(rg + 1) * SG, :]
        d2q = (
            xq_sg[:, 0:1] * xq_sg[:, 0:1] + xq_sg[:, 1:2] * xq_sg[:, 1:2]
        ) + xq_sg[:, 2:3] * xq_sg[:, 2:3]
        dot = jnp.dot(xq_sg, xpt, preferred_element_type=jnp.float32)
        dmat = (d2q + d2p) - 2.0 * dot
        dmat_ref[pl.ds(rg * SG, SG), :] = dmat
        v = [jnp.full((SG, 128), inf, jnp.float32) for _ in range(S)]
        gi = [jnp.full((SG, 128), bigi, jnp.int32) for _ in range(S)]
        for c in range(NC_):
            t = dmat[:, c * 128:(c + 1) * 128]
            ti = liota + (c * 128)
            for j in range(S):
                lt = t < v[j]
                nv = jnp.where(lt, t, v[j])
                ni = jnp.where(lt, ti, gi[j])
                if j < S - 1:
                    t = jnp.where(lt, v[j], t)
                    ti = jnp.where(lt, gi[j], ti)
                v[j], gi[j] = nv, ni
        res = (
            lax.broadcasted_iota(jnp.int32, (SG, KP), 0) + (rg * SG + base)
        )
        flag = jnp.zeros((SG, 128), jnp.int32)
        for k in range(K):
            m = jnp.min(v[0], axis=1, keepdims=True)
            ik = jnp.min(jnp.where(v[0] == m, gi[0], bigi), axis=1, keepdims=True)
            res = jnp.where(kiota == k, ik + base, res)
            ext = (v[0] == m) & (gi[0] == ik)
            for j in range(S - 1):
                v[j] = jnp.where(ext, v[j + 1], v[j])
                gi[j] = jnp.where(ext, gi[j + 1], gi[j])
            v[S - 1] = jnp.where(ext, inf, v[S - 1])
            gi[S - 1] = jnp.where(ext, bigi, gi[S - 1])
            flag = flag | jnp.where(ext & (v[0] == inf), 1, 0)
        out_ref[0, pl.ds(rg * SG, SG), :] = res
        flags_ref[rg] = jnp.max(flag)

    for rg_ in range(NT // SG):
        subgroup(rg_)

    giota = lax.broadcasted_iota(jnp.int32, (SG, N), 1)

    def fb_subgroup(rg, _):
        @pl.when(flags_ref[rg] > 0)
        def _():
            def fb_k(k, res):
                d = dmat_ref[pl.ds(rg * SG, SG), :]
                m = jnp.min(d, axis=1, keepdims=True)
                ik = jnp.min(jnp.where(d == m, giota, bigi), axis=1, keepdims=True)
                dmat_ref[pl.ds(rg * SG, SG), :] = jnp.where(giota == ik, inf, d)
                return jnp.where(kiota == k, ik + base, res)

            res = lax.fori_loop(0, K, fb_k, jnp.zeros((SG, KP), jnp.int32))
            out_ref[0, pl.ds(rg * SG, SG), :] = res

        return 0

    lax.fori_loop(0, NT // SG, fb_subgroup, 0)


def _run_topk(x_vb):
    xvt = x_vb.T
    return pl.pallas_call(
        _topk_kernel,
        grid=(T,),
        in_specs=[
            pl.BlockSpec((NT, 3), lambda t: (t, 0)),
            pl.BlockSpec((3, N), lambda t: (0, 0)),
        ],
        out_specs=pl.BlockSpec((1, NT, KP), lambda t: (t, 0, 0)),
        out_shape=jax.ShapeDtypeStruct((T, NT, KP), jnp.int32),
        scratch_shapes=[
            pltpu.VMEM((NT, N), jnp.float32),
            pltpu.SMEM((NT // SG,), jnp.int32),
        ],
    )(x_vb, xvt)


NW = 32
ROWS = N * K
PW = ROWS // NW
CH = 288
NCH = PW // CH


def _gather_body(table_hbm, idx_hbm, out_hbm, idx_v, rows0, rows1, sem0, sem1):
    wid = lax.axis_index("s") * 2 + lax.axis_index("c")
    base = wid * PW
    pltpu.sync_copy(idx_hbm.at[pl.ds(base, PW)], idx_v)
    bufs = (rows0, rows1)
    sems = (sem0, sem1)
    pltpu.async_copy(table_hbm.at[idx_v.at[pl.ds(0, CH)]], rows0, sem0)
    for ci in range(NCH):
        if ci + 1 < NCH:
            pltpu.async_copy(
                table_hbm.at[idx_v.at[pl.ds((ci + 1) * CH, CH)]],
                bufs[(ci + 1) % 2],
                sems[(ci + 1) % 2],
            )
        pltpu.make_async_copy(
            table_hbm.at[idx_v.at[pl.ds(ci * CH, CH)]],
            bufs[ci % 2],
            sems[ci % 2],
        ).wait()
        pltpu.sync_copy(bufs[ci % 2], out_hbm.at[pl.ds(base + ci * CH, CH)])


@functools.cache
def _make_gather():
    return pl.kernel(
        _gather_body,
        mesh=plsc.VectorSubcoreMesh(core_axis_name="c", subcore_axis_name="s"),
        out_type=jax.ShapeDtypeStruct((ROWS, D), jnp.float32),
        scratch_types=[
            pltpu.VMEM((PW,), jnp.int32),
            pltpu.VMEM((CH, D), jnp.float32),
            pltpu.VMEM((CH, D), jnp.float32),
            pltpu.SemaphoreType.DMA,
            pltpu.SemaphoreType.DMA,
        ],
    )


def _mm_kernel(flat_ref, wt_ref, b_ref, out_ref):
    out_ref[...] = (
        jnp.dot(flat_ref[...], wt_ref[...], preferred_element_type=jnp.float32)
        + b_ref[...]
    )


def _run_mm(flat, wt, bias):
    return pl.pallas_call(
        _mm_kernel,
        grid=(T,),
        in_specs=[
            pl.BlockSpec((NT, K * D), lambda i: (i, 0)),
            pl.BlockSpec((K * D, E), lambda i: (0, 0)),
            pl.BlockSpec((1, E), lambda i: (0, 0)),
        ],
        out_specs=pl.BlockSpec((NT, E), lambda i: (i, 0)),
        out_shape=jax.ShapeDtypeStruct((N, E), jnp.float32),
    )(flat, wt, bias)


def kernel(x, x_v, W, b):
    wt = W.T
    bias = b.reshape(1, E)
    gather = _make_gather()
    outs = []
    for bb in range(B):
        idxg = _run_topk(x_v[bb])
        idx = idxg[..., :K].reshape(ROWS)
        g = gather(x[bb], idx)
        flat = g.reshape(N, K * D)
        outs.append(_run_mm(flat, wt, bias))
    return jnp.stack(outs)


# speedup vs baseline: 1.1771x; 1.0442x over previous
import functools

import jax
import jax.numpy as jnp
from jax import lax
from jax.experimental import pallas as pl
from jax.experimental.pallas import tpu as pltpu
from jax.experimental.pallas import tpu_sc as plsc

B, N, D, K, E = 4, 4096, 128, 27, 256
NT = 256
T = N // NT
SG = 16
KP = 32


S = 5
NC_ = N // 128


def _topk_kernel(xv_ref, xvt_ref, out_ref, dmat_ref, flags_ref):
    xq = xv_ref[...]
    xpt = xvt_ref[...]
    d2p = (xpt[0:1, :] * xpt[0:1, :] + xpt[1:2, :] * xpt[1:2, :]) + xpt[2:3, :] * xpt[2:3, :]

    base = 0
    inf = jnp.float32(jnp.inf)
    bigi = jnp.int32(N)
    liota = lax.broadcasted_iota(jnp.int32, (SG, 128), 1)
    kiota = lax.broadcasted_iota(jnp.int32, (SG, KP), 1)

    def subgroup(rg):
        xq_sg = xq[rg * SG:(rg + 1) * SG, :]
        d2q = (
            xq_sg[:, 0:1] * xq_sg[:, 0:1] + xq_sg[:, 1:2] * xq_sg[:, 1:2]
        ) + xq_sg[:, 2:3] * xq_sg[:, 2:3]
        dot = jnp.dot(xq_sg, xpt, preferred_element_type=jnp.float32)
        dmat = (d2q + d2p) - 2.0 * dot
        dmat_ref[pl.ds(rg * SG, SG), :] = dmat
        v = [jnp.full((SG, 128), inf, jnp.float32) for _ in range(S)]
        gi = [jnp.full((SG, 128), bigi, jnp.int32) for _ in range(S)]
        for c in range(NC_):
            t = dmat[:, c * 128:(c + 1) * 128]
            ti = liota + (c * 128)
            for j in range(S):
                lt = t < v[j]
                nv = jnp.where(lt, t, v[j])
                ni = jnp.where(lt, ti, gi[j])
                if j < S - 1:
                    t = jnp.where(lt, v[j], t)
                    ti = jnp.where(lt, gi[j], ti)
                v[j], gi[j] = nv, ni
        res = (
            lax.broadcasted_iota(jnp.int32, (SG, KP), 0) + (rg * SG + base)
        )
        flag = jnp.zeros((SG, 128), jnp.int32)
        for k in range(K):
            m = jnp.min(v[0], axis=1, keepdims=True)
            ik = jnp.min(jnp.where(v[0] == m, gi[0], bigi), axis=1, keepdims=True)
            res = jnp.where(kiota == k, ik + base, res)
            ext = (v[0] == m) & (gi[0] == ik)
            for j in range(S - 1):
                v[j] = jnp.where(ext, v[j + 1], v[j])
                gi[j] = jnp.where(ext, gi[j + 1], gi[j])
            v[S - 1] = jnp.where(ext, inf, v[S - 1])
            gi[S - 1] = jnp.where(ext, bigi, gi[S - 1])
            flag = flag | jnp.where(ext & (v[0] == inf), 1, 0)
        out_ref[0, pl.ds(rg * SG, SG), :] = res
        flags_ref[rg] = jnp.max(flag)

    for rg_ in range(NT // SG):
        subgroup(rg_)

    giota = lax.broadcasted_iota(jnp.int32, (SG, N), 1)

    def fb_subgroup(rg, _):
        @pl.when(flags_ref[rg] > 0)
        def _():
            def fb_k(k, res):
                d = dmat_ref[pl.ds(rg * SG, SG), :]
                m = jnp.min(d, axis=1, keepdims=True)
                ik = jnp.min(jnp.where(d == m, giota, bigi), axis=1, keepdims=True)
                dmat_ref[pl.ds(rg * SG, SG), :] = jnp.where(giota == ik, inf, d)
                return jnp.where(kiota == k, ik + base, res)

            res = lax.fori_loop(0, K, fb_k, jnp.zeros((SG, KP), jnp.int32))
            out_ref[0, pl.ds(rg * SG, SG), :] = res

        return 0

    lax.fori_loop(0, NT // SG, fb_subgroup, 0)


def _run_topk(x_vb):
    xvt = x_vb.T
    return pl.pallas_call(
        _topk_kernel,
        grid=(T,),
        in_specs=[
            pl.BlockSpec((NT, 3), lambda t: (t, 0)),
            pl.BlockSpec((3, N), lambda t: (0, 0)),
        ],
        out_specs=pl.BlockSpec((1, NT, KP), lambda t: (t, 0, 0)),
        out_shape=jax.ShapeDtypeStruct((T, NT, KP), jnp.int32),
        scratch_shapes=[
            pltpu.VMEM((NT, N), jnp.float32),
            pltpu.SMEM((NT // SG,), jnp.int32),
        ],
    )(x_vb, xvt)


NW = 32
ROWS = N * K
PW = ROWS // NW
CH = 288
NCH = PW // CH


def _gather_body(table_hbm, idx_hbm, out_hbm, idx_v, rows0, rows1, sem0, sem1):
    wid = lax.axis_index("s") * 2 + lax.axis_index("c")
    base = wid * PW
    pltpu.sync_copy(idx_hbm.at[pl.ds(base, PW)], idx_v)
    bufs = (rows0, rows1)
    sems = (sem0, sem1)
    pltpu.async_copy(table_hbm.at[idx_v.at[pl.ds(0, CH)]], rows0, sem0)
    for ci in range(NCH):
        if ci + 1 < NCH:
            pltpu.async_copy(
                table_hbm.at[idx_v.at[pl.ds((ci + 1) * CH, CH)]],
                bufs[(ci + 1) % 2],
                sems[(ci + 1) % 2],
            )
        pltpu.make_async_copy(
            table_hbm.at[idx_v.at[pl.ds(ci * CH, CH)]],
            bufs[ci % 2],
            sems[ci % 2],
        ).wait()
        pltpu.sync_copy(bufs[ci % 2], out_hbm.at[pl.ds(base + ci * CH, CH)])


@functools.cache
def _make_gather():
    return pl.kernel(
        _gather_body,
        mesh=plsc.VectorSubcoreMesh(core_axis_name="c", subcore_axis_name="s"),
        out_type=jax.ShapeDtypeStruct((ROWS, D), jnp.float32),
        scratch_types=[
            pltpu.VMEM((PW,), jnp.int32),
            pltpu.VMEM((CH, D), jnp.float32),
            pltpu.VMEM((CH, D), jnp.float32),
            pltpu.SemaphoreType.DMA,
            pltpu.SemaphoreType.DMA,
        ],
    )


def _mm_kernel(flat_ref, wt_ref, b_ref, out_ref):
    out_ref[...] = (
        jnp.dot(flat_ref[...], wt_ref[...], preferred_element_type=jnp.float32)
        + b_ref[...]
    )


def _run_mm(flat, wt, bias):
    return pl.pallas_call(
        _mm_kernel,
        grid=(T,),
        in_specs=[
            pl.BlockSpec((NT, K * D), lambda i: (i, 0)),
            pl.BlockSpec((K * D, E), lambda i: (0, 0)),
            pl.BlockSpec((1, E), lambda i: (0, 0)),
        ],
        out_specs=pl.BlockSpec((NT, E), lambda i: (i, 0)),
        out_shape=jax.ShapeDtypeStruct((N, E), jnp.float32),
    )(flat, wt, bias)


def kernel(x, x_v, W, b):
    wt = W.T
    bias = b.reshape(1, E)
    gather = _make_gather()
    outs = []
    for bb in range(B):
        idxg = _run_topk(x_v[bb])
        idx = idxg[..., :K].reshape(ROWS)
        g = gather(x[bb], idx)
        flat = g.reshape(N, K * D)
        outs.append(_run_mm(flat, wt, bias))
    return jnp.stack(outs)
